# Initial kernel scaffold; baseline (speedup 1.0000x reference)
import jax
import jax.numpy as jnp
from jax.experimental import pallas as pl


def kernel(user_id, launch_seq, user_table, launch_table, W_ih, W_hh, b_ih, b_hh, fc_W, fc_b):
    raise NotImplementedError("write your pallas kernel here")


# speedup vs baseline: 10.0645x; 10.0645x over previous
import functools

import jax
import jax.numpy as jnp
from jax import lax
from jax.experimental import pallas as pl
from jax.experimental.pallas import tpu as pltpu
from jax.experimental.pallas import tpu_sc as plsc


def _make_user_gather(V, D, B):
    info = plsc.get_sparse_core_info()
    NC, NS = info.num_cores, info.num_subcores
    NW = NC * NS
    assert B % (8 * NW) == 0 and D % info.num_lanes == 0
    b_per_w = B // NW
    mesh = plsc.VectorSubcoreMesh(core_axis_name="c", subcore_axis_name="s")

    @functools.partial(
        pl.kernel,
        mesh=mesh,
        compiler_params=pltpu.CompilerParams(use_tc_tiling_on_sc=False),
        out_type=jax.ShapeDtypeStruct((B, D), jnp.float32),
        scratch_types=[
            pltpu.VMEM((b_per_w,), jnp.int32),
            pltpu.VMEM((b_per_w, D), jnp.float32),
            pltpu.SemaphoreType.DMA,
        ],
    )
    def gather_k(table_hbm, idx_hbm, out_hbm, idx_v, rows_v, sem):
        wid = lax.axis_index("s") * NC + lax.axis_index("c")
        base = wid * b_per_w
        pltpu.sync_copy(idx_hbm.at[pl.ds(base, b_per_w)], idx_v)
        pltpu.async_copy(table_hbm.at[idx_v], rows_v, sem).wait()
        pltpu.sync_copy(rows_v, out_hbm.at[pl.ds(base, b_per_w)])

    return gather_k


def _gru_body(ls_ref, ue_ref, ltT_ref, wih_ref, whh_ref, bih_ref, bhh_ref,
              fcw_ref, fcb_ref, out_ref, h_ref, acc_ref):
    L = ls_ref.shape[0]
    H = h_ref.shape[0]
    Gt = jnp.dot(wih_ref[...], ltT_ref[...],
                 preferred_element_type=jnp.float32) + bih_ref[...]
    Gtb = jnp.concatenate([Gt[0:2 * H] + bhh_ref[0:2 * H], Gt[2 * H:3 * H]],
                          axis=0)
    whh = whh_ref[...]
    bhh_n = bhh_ref[2 * H:3 * H]
    iota3 = lax.broadcasted_iota(jnp.int32, (3, 1), 0)

    h_ref[...] = jnp.zeros_like(h_ref)
    acc_ref[...] = jnp.zeros_like(acc_ref)

    def step(t, carry):
        ls_row = ls_ref[pl.ds(t, 1), :]
        oh = jnp.equal(ls_row, iota3).astype(jnp.float32)
        h = h_ref[...]
        gh = jnp.dot(whh, h, preferred_element_type=jnp.float32)
        gi = jnp.dot(Gtb, oh, preferred_element_type=jnp.float32)
        rz = jax.nn.sigmoid(gh[0:2 * H] + gi[0:2 * H])
        r = rz[0:H]
        z = rz[H:2 * H]
        hn = gh[2 * H:3 * H] + bhh_n
        n = jnp.tanh(gi[2 * H:3 * H] + r * hn)
        h_new = n + z * (h - n)
        h_ref[...] = h_new
        acc_ref[...] += h_new
        return carry

    lax.fori_loop(0, L, step, 0, unroll=2)

    seq_feat = acc_ref[...] * (1.0 / L)
    wu = fcw_ref[0:H, :]
    ws = fcw_ref[H:2 * H, :]
    contrib = wu * ue_ref[...] + ws * seq_feat
    out_ref[...] = jnp.sum(contrib, axis=0, keepdims=True) + fcb_ref[...]


def _gru_call(ls_t, ue_t, ltT, W_ih, W_hh, b_ih2, b_hh2, fcwT, fcb2):
    L, B = ls_t.shape
    H = ue_t.shape[0]
    return pl.pallas_call(
        _gru_body,
        out_shape=jax.ShapeDtypeStruct((1, B), jnp.float32),
        scratch_shapes=[
            pltpu.VMEM((H, B), jnp.float32),
            pltpu.VMEM((H, B), jnp.float32),
        ],
    )(ls_t, ue_t, ltT, W_ih, W_hh, b_ih2, b_hh2, fcwT, fcb2)


def kernel(user_id, launch_seq, user_table, launch_table, W_ih, W_hh, b_ih,
           b_hh, fc_W, fc_b):
    B, L = launch_seq.shape
    V, E = user_table.shape
    H = W_hh.shape[1]

    gather = _make_user_gather(V, E, B)
    user_rows = gather(user_table, user_id.astype(jnp.int32))

    ue_t = user_rows.T
    ls_t = launch_seq.T
    ltT = launch_table.T
    b_ih2 = b_ih.reshape(3 * H, 1)
    b_hh2 = b_hh.reshape(3 * H, 1)
    fcwT = fc_W.reshape(2 * H, 1)
    fcb2 = fc_b.reshape(1, 1)

    out_row = _gru_call(ls_t, ue_t, ltT, W_ih, W_hh, b_ih2, b_hh2, fcwT, fcb2)
    return out_row.reshape(B, 1)


# speedup vs baseline: 11.4190x; 1.1346x over previous
import functools

import jax
import jax.numpy as jnp
from jax import lax
from jax.experimental import pallas as pl
from jax.experimental.pallas import tpu as pltpu
from jax.experimental.pallas import tpu_sc as plsc


def _make_user_gather(V, D, B):
    info = plsc.get_sparse_core_info()
    NC, NS = info.num_cores, info.num_subcores
    NW = NC * NS
    assert B % (8 * NW) == 0 and D % info.num_lanes == 0
    b_per_w = B // NW
    mesh = plsc.VectorSubcoreMesh(core_axis_name="c", subcore_axis_name="s")

    @functools.partial(
        pl.kernel,
        mesh=mesh,
        compiler_params=pltpu.CompilerParams(use_tc_tiling_on_sc=False),
        out_type=jax.ShapeDtypeStruct((B, D), jnp.float32),
        scratch_types=[
            pltpu.VMEM((b_per_w,), jnp.int32),
            pltpu.VMEM((b_per_w, D), jnp.float32),
            pltpu.SemaphoreType.DMA,
        ],
    )
    def gather_k(table_hbm, idx_hbm, out_hbm, idx_v, rows_v, sem):
        wid = lax.axis_index("s") * NC + lax.axis_index("c")
        base = wid * b_per_w
        pltpu.sync_copy(idx_hbm.at[pl.ds(base, b_per_w)], idx_v)
        pltpu.async_copy(table_hbm.at[idx_v], rows_v, sem).wait()
        pltpu.sync_copy(rows_v, out_hbm.at[pl.ds(base, b_per_w)])

    return gather_k


def _gru_body(ls_ref, ue_ref, ltT_ref, wih_ref, whh_ref, bih_ref, bhh_ref,
              fcw_ref, fcb_ref, out_ref, s_ref, acc_ref):
    L = ls_ref.shape[0]
    H = ue_ref.shape[0]
    Gt = jnp.dot(wih_ref[...], ltT_ref[...],
                 preferred_element_type=jnp.float32) + bih_ref[...]
    whh = whh_ref[...]
    A = jnp.concatenate([
        jnp.concatenate([whh[0:2 * H], Gt[0:2 * H] + bhh_ref[0:2 * H]],
                        axis=1),
        jnp.concatenate([whh[2 * H:3 * H],
                         jnp.broadcast_to(bhh_ref[2 * H:3 * H], (H, 3))],
                        axis=1),
        jnp.concatenate([jnp.zeros((H, H), jnp.float32), Gt[2 * H:3 * H]],
                        axis=1),
    ], axis=0)
    iota3 = lax.broadcasted_iota(jnp.int32, (3, 1), 0)

    s_ref[...] = jnp.zeros_like(s_ref)
    acc_ref[...] = jnp.zeros_like(acc_ref)

    def step(t, carry):
        ls_row = ls_ref[pl.ds(t, 1), :]
        s_ref[H:H + 3, :] = jnp.equal(ls_row, iota3).astype(jnp.float32)
        gates = jnp.dot(A, s_ref[...],
                        preferred_element_type=jnp.float32)
        rz = 0.5 * jnp.tanh(0.5 * gates[0:2 * H]) + 0.5
        r = rz[0:H]
        z = rz[H:2 * H]
        n = jnp.tanh(gates[3 * H:4 * H] + r * gates[2 * H:3 * H])
        h = s_ref[0:H, :]
        h_new = n + z * (h - n)
        s_ref[0:H, :] = h_new
        acc_ref[...] += h_new
        return carry

    lax.fori_loop(0, L, step, 0, unroll=4)

    seq_feat = acc_ref[...] * (1.0 / L)
    wu = fcw_ref[0:H, :]
    ws = fcw_ref[H:2 * H, :]
    contrib = wu * ue_ref[...] + ws * seq_feat
    out_ref[...] = jnp.sum(contrib, axis=0, keepdims=True) + fcb_ref[...]


def _gru_call(ls_t, ue_t, ltT, W_ih, W_hh, b_ih2, b_hh2, fcwT, fcb2):
    L, B = ls_t.shape
    H = ue_t.shape[0]
    return pl.pallas_call(
        _gru_body,
        out_shape=jax.ShapeDtypeStruct((1, B), jnp.float32),
        scratch_shapes=[
            pltpu.VMEM((H + 3, B), jnp.float32),
            pltpu.VMEM((H, B), jnp.float32),
        ],
    )(ls_t, ue_t, ltT, W_ih, W_hh, b_ih2, b_hh2, fcwT, fcb2)


def kernel(user_id, launch_seq, user_table, launch_table, W_ih, W_hh, b_ih,
           b_hh, fc_W, fc_b):
    B, L = launch_seq.shape
    V, E = user_table.shape
    H = W_hh.shape[1]

    gather = _make_user_gather(V, E, B)
    user_rows = gather(user_table, user_id.astype(jnp.int32))

    ue_t = user_rows.T
    ls_t = launch_seq.T
    ltT = launch_table.T
    b_ih2 = b_ih.reshape(3 * H, 1)
    b_hh2 = b_hh.reshape(3 * H, 1)
    fcwT = fc_W.reshape(2 * H, 1)
    fcb2 = fc_b.reshape(1, 1)

    out_row = _gru_call(ls_t, ue_t, ltT, W_ih, W_hh, b_ih2, b_hh2, fcwT, fcb2)
    return out_row.reshape(B, 1)
